# baseline (speedup 1.0000x reference)
import jax
import jax.numpy as jnp
from jax.experimental import pallas as pl
from jax.experimental.pallas import tpu as pltpu

M_CAP = 1024
DIM = 512
BATCH = 64
INIT_SIZE = M_CAP - BATCH
A_T = 0.35
H_T = 0.3


def _gwr_body(scal_ref, data_ref, V_in, h_in, E_in, V_out, h_out, acts_out, E_scr):
    V_out[...] = V_in[...]
    h_out[...] = h_in[...]
    E_scr[...] = E_in[...]

    eps_b = scal_ref[0]
    eps_n = scal_ref[1]
    tau_b = scal_ref[2]
    tau_n = scal_ref[3]
    kappa = scal_ref[4]

    i8 = (jax.lax.broadcasted_iota(jnp.int32, (8, 128), 0) * 128
          + jax.lax.broadcasted_iota(jnp.int32, (8, 128), 1))
    irow = jax.lax.broadcasted_iota(jnp.int32, (1, M_CAP), 1)
    iacts = jax.lax.broadcasted_iota(jnp.int32, (1, BATCH), 1)

    def step(t, carry):
        size, acts = carry
        x = data_ref[pl.ds(t, 1), :]
        V = V_out[...]
        diff = x - V
        d2 = jnp.sum(diff * diff, axis=1, keepdims=True)
        d8 = jnp.reshape(jnp.reshape(d2, (1, M_CAP)), (8, 128))
        d = jnp.sqrt(d8 + 1e-12)
        d = jnp.where(i8 < size, d, jnp.inf)
        m = jnp.min(d)
        b = jnp.min(jnp.where(d == m, i8, M_CAP))
        d_wo = jnp.where(i8 == b, jnp.inf, d)
        m2 = jnp.min(d_wo)
        s = jnp.min(jnp.where(d_wo == m2, i8, M_CAP))
        a = jnp.exp(-m)
        h8 = h_out[...]
        hb = jnp.sum(jnp.where(i8 == b, h8, 0.0))
        insert = (a < A_T) & (hb < H_T) & (size < M_CAP)

        @pl.when(insert)
        def _():
            r = size
            Vb = V_out[pl.ds(b, 1), :]
            V_out[pl.ds(r, 1), :] = (x + Vb) * 0.5
            h_out[...] = jnp.where(i8 == r, 1.0, h8)
            erb = E_scr[pl.ds(b, 1), :]
            E_scr[pl.ds(b, 1), :] = jnp.where(
                irow == s, -1, jnp.where(irow == r, 0, erb))
            ers = E_scr[pl.ds(s, 1), :]
            E_scr[pl.ds(s, 1), :] = jnp.where(
                irow == b, -1, jnp.where(irow == r, 0, ers))
            E_scr[pl.ds(r, 1), :] = jnp.where(
                (irow == b) | (irow == s), 0, -1).astype(jnp.int32)

        @pl.when(jnp.logical_not(insert))
        def _():
            er2 = jnp.where(irow == s, 0, E_scr[pl.ds(b, 1), :])
            E_scr[pl.ds(b, 1), :] = er2
            ers = E_scr[pl.ds(s, 1), :]
            E_scr[pl.ds(s, 1), :] = jnp.where(irow == b, 0, ers)
            nb_r = (er2 > -1) & (irow != b)
            nbf_row = nb_r.astype(jnp.float32)
            nb8 = jnp.reshape(nbf_row, (8, 128))
            c_row = nbf_row * (eps_n * jnp.reshape(h8, (1, M_CAP)))
            c_col = jnp.reshape(c_row, (M_CAP, 1))

            hb_new = hb + tau_b * kappa * (1.0 - hb) - tau_b
            Vb = V_out[pl.ds(b, 1), :]
            Vb_new = Vb + eps_b * hb * (x - Vb)
            slab_any = jnp.max(nb8, axis=1, keepdims=True)
            ir8 = jax.lax.broadcasted_iota(jnp.int32, (8, 1), 0)
            for rblk in range(8):
                has = jnp.sum(jnp.where(ir8 == rblk, slab_any, 0.0)) > 0.0

                @pl.when(has)
                def _(rblk=rblk):
                    Vs = V_out[pl.ds(rblk * 128, 128), :]
                    cs = c_col[rblk * 128:(rblk + 1) * 128, :]
                    V_out[pl.ds(rblk * 128, 128), :] = Vs + cs * (x - Vs)

            V_out[pl.ds(b, 1), :] = Vb_new
            hn8 = h8 + nb8 * (tau_n * kappa * (1.0 - h8) - tau_n)
            h_out[...] = jnp.where(i8 == b, hb_new, hn8)

        acts = jnp.where(iacts == t, a, acts)
        size = jnp.where(insert, size + jnp.int32(1), size)
        return size, acts

    size0 = jnp.int32(INIT_SIZE)
    acts0 = jnp.zeros((1, BATCH), jnp.float32)
    _, acts_f = jax.lax.fori_loop(0, BATCH, step, (size0, acts0))
    acts_out[...] = acts_f


def kernel(it, data, V, h, E, eps_b, eps_n, tau_b, tau_n, kappa):
    scal = jnp.stack([eps_b, eps_n, tau_b, tau_n, kappa]).astype(jnp.float32)
    Vf, hf, acts = pl.pallas_call(
        _gwr_body,
        out_shape=[
            jax.ShapeDtypeStruct((M_CAP, DIM), jnp.float32),
            jax.ShapeDtypeStruct((8, 128), jnp.float32),
            jax.ShapeDtypeStruct((1, BATCH), jnp.float32),
        ],
        in_specs=[
            pl.BlockSpec(memory_space=pltpu.SMEM),
            pl.BlockSpec(memory_space=pltpu.VMEM),
            pl.BlockSpec(memory_space=pltpu.VMEM),
            pl.BlockSpec(memory_space=pltpu.VMEM),
            pl.BlockSpec(memory_space=pltpu.VMEM),
        ],
        out_specs=[
            pl.BlockSpec(memory_space=pltpu.VMEM),
            pl.BlockSpec(memory_space=pltpu.VMEM),
            pl.BlockSpec(memory_space=pltpu.VMEM),
        ],
        scratch_shapes=[pltpu.VMEM((M_CAP, M_CAP), jnp.int32)],
    )(scal, data, V, h.reshape(8, 128), E)
    return Vf, hf.reshape(M_CAP), acts.reshape(BATCH)


# speedup vs baseline: 1.3060x; 1.3060x over previous
import jax
import jax.numpy as jnp
from jax.experimental import pallas as pl
from jax.experimental.pallas import tpu as pltpu

M_CAP = 1024
DIM = 512
BATCH = 64
INIT_SIZE = M_CAP - BATCH
A_T = 0.35
H_T = 0.3


def _gwr_body(scal_ref, data_ref, V_in, h_in, E_in, V_out, h_out, acts_out,
              E_scr, d8_scr):
    V_out[...] = V_in[...]
    h_out[...] = h_in[...]
    E_scr[...] = E_in[...]

    eps_b = scal_ref[0]
    eps_n = scal_ref[1]
    tau_b = scal_ref[2]
    tau_n = scal_ref[3]
    kappa = scal_ref[4]

    i8 = (jax.lax.broadcasted_iota(jnp.int32, (8, 128), 0) * 128
          + jax.lax.broadcasted_iota(jnp.int32, (8, 128), 1))
    irow = jax.lax.broadcasted_iota(jnp.int32, (1, M_CAP), 1)
    iacts = jax.lax.broadcasted_iota(jnp.int32, (1, BATCH), 1)

    def dist8(xv, Vv):
        d2 = jnp.sum((xv - Vv) * (xv - Vv), axis=1, keepdims=True)
        return jnp.reshape(jnp.reshape(d2, (1, M_CAP)), (8, 128))

    x0 = data_ref[pl.ds(0, 1), :]
    d8_0 = dist8(x0, V_out[...])

    def step(t, carry):
        size, acts, d8 = carry
        x = data_ref[pl.ds(t, 1), :]
        tn = jnp.minimum(t + 1, BATCH - 1)
        xn = data_ref[pl.ds(tn, 1), :]
        d = jnp.sqrt(d8 + 1e-12)
        d = jnp.where(i8 < size, d, jnp.inf)
        m = jnp.min(d)
        b = jnp.min(jnp.where(d == m, i8, M_CAP))
        d_wo = jnp.where(i8 == b, jnp.inf, d)
        m2 = jnp.min(d_wo)
        s = jnp.min(jnp.where(d_wo == m2, i8, M_CAP))
        a = jnp.exp(-m)
        h8 = h_out[...]
        hb = jnp.sum(jnp.where(i8 == b, h8, 0.0))
        insert = (a < A_T) & (hb < H_T) & (size < M_CAP)

        @pl.when(insert)
        def _():
            r = size
            Vb = V_out[pl.ds(b, 1), :]
            V_out[pl.ds(r, 1), :] = (x + Vb) * 0.5
            h_out[...] = jnp.where(i8 == r, 1.0, h8)
            erb = E_scr[pl.ds(b, 1), :]
            E_scr[pl.ds(b, 1), :] = jnp.where(
                irow == s, -1, jnp.where(irow == r, 0, erb))
            ers = E_scr[pl.ds(s, 1), :]
            E_scr[pl.ds(s, 1), :] = jnp.where(
                irow == b, -1, jnp.where(irow == r, 0, ers))
            E_scr[pl.ds(r, 1), :] = jnp.where(
                (irow == b) | (irow == s), 0, -1).astype(jnp.int32)
            d8_scr[...] = dist8(xn, V_out[...])

        @pl.when(jnp.logical_not(insert))
        def _():
            er2 = jnp.where(irow == s, 0, E_scr[pl.ds(b, 1), :])
            E_scr[pl.ds(b, 1), :] = er2
            ers = E_scr[pl.ds(s, 1), :]
            E_scr[pl.ds(s, 1), :] = jnp.where(irow == b, 0, ers)
            nb_r = (er2 > -1) & (irow != b)
            nbf_row = nb_r.astype(jnp.float32)
            nb8 = jnp.reshape(nbf_row, (8, 128))
            c_row = nbf_row * (eps_n * jnp.reshape(h8, (1, M_CAP)))
            c_col = jnp.reshape(c_row, (M_CAP, 1))

            hb_new = hb + tau_b * kappa * (1.0 - hb) - tau_b
            Vb = V_out[pl.ds(b, 1), :]
            Vb_new = Vb + eps_b * hb * (x - Vb)
            Vv = V_out[...]
            Vnew = Vv + c_col * (x - Vv)
            V_out[...] = Vnew
            d8n = dist8(xn, Vnew)
            V_out[pl.ds(b, 1), :] = Vb_new
            dbn = xn - Vb_new
            d2b = jnp.sum(dbn * dbn)
            d8_scr[...] = jnp.where(i8 == b, d2b, d8n)
            hn8 = h8 + nb8 * (tau_n * kappa * (1.0 - h8) - tau_n)
            h_out[...] = jnp.where(i8 == b, hb_new, hn8)

        acts = jnp.where(iacts == t, a, acts)
        size = jnp.where(insert, size + jnp.int32(1), size)
        return size, acts, d8_scr[...]

    size0 = jnp.int32(INIT_SIZE)
    acts0 = jnp.zeros((1, BATCH), jnp.float32)
    _, acts_f, _ = jax.lax.fori_loop(0, BATCH, step, (size0, acts0, d8_0))
    acts_out[...] = acts_f


def kernel(it, data, V, h, E, eps_b, eps_n, tau_b, tau_n, kappa):
    scal = jnp.stack([eps_b, eps_n, tau_b, tau_n, kappa]).astype(jnp.float32)
    Vf, hf, acts = pl.pallas_call(
        _gwr_body,
        out_shape=[
            jax.ShapeDtypeStruct((M_CAP, DIM), jnp.float32),
            jax.ShapeDtypeStruct((8, 128), jnp.float32),
            jax.ShapeDtypeStruct((1, BATCH), jnp.float32),
        ],
        in_specs=[
            pl.BlockSpec(memory_space=pltpu.SMEM),
            pl.BlockSpec(memory_space=pltpu.VMEM),
            pl.BlockSpec(memory_space=pltpu.VMEM),
            pl.BlockSpec(memory_space=pltpu.VMEM),
            pl.BlockSpec(memory_space=pltpu.VMEM),
        ],
        out_specs=[
            pl.BlockSpec(memory_space=pltpu.VMEM),
            pl.BlockSpec(memory_space=pltpu.VMEM),
            pl.BlockSpec(memory_space=pltpu.VMEM),
        ],
        scratch_shapes=[
            pltpu.VMEM((M_CAP, M_CAP), jnp.int32),
            pltpu.VMEM((8, 128), jnp.float32),
        ],
    )(scal, data, V, h.reshape(8, 128), E)
    return Vf, hf.reshape(M_CAP), acts.reshape(BATCH)
